# baseline (speedup 1.0000x reference)
import jax
import jax.numpy as jnp
from jax import lax
from jax.experimental import pallas as pl
from jax.experimental.pallas import tpu as pltpu
from jax.experimental.pallas import tpu_sc as plsc

NUM_NODES = 10000
NUM_EDGES = 320000
NUM_REL = 500
H = 64
ZD = 2 * H
NC = 2
NS = 16
L = 16
NW = NC * NS
EPW = NUM_EDGES // NW
CHUNK = 80
NCHUNK = EPW // CHUNK
GROUPS = CHUNK // L

def _score_body(zp_hbm, hidx_hbm, tidx_hbm, et_hbm, rel_hbm, reli_hbm,
                out_hbm,
                rel_v, reli_v, hidx_all, tidx_all,
                head_v0, head_v1, tail_v0, tail_v1,
                et_all, scores_v,
                sem_g0, sem_g1):
    wid = lax.axis_index("s") * NC + lax.axis_index("c")
    base0 = wid * EPW
    pltpu.make_async_copy(rel_hbm, rel_v, sem_g0).start()
    pltpu.make_async_copy(reli_hbm, reli_v, sem_g0).start()
    pltpu.make_async_copy(hidx_hbm.at[pl.ds(base0, EPW)], hidx_all, sem_g0).start()
    pltpu.make_async_copy(tidx_hbm.at[pl.ds(base0, EPW)], tidx_all, sem_g0).start()
    pltpu.make_async_copy(et_hbm.at[pl.ds(base0, EPW)], et_all, sem_g0).start()
    pltpu.make_async_copy(rel_hbm, rel_v, sem_g0).wait()
    pltpu.make_async_copy(reli_hbm, reli_v, sem_g0).wait()
    pltpu.make_async_copy(hidx_hbm.at[pl.ds(base0, EPW)], hidx_all, sem_g0).wait()
    pltpu.make_async_copy(tidx_hbm.at[pl.ds(base0, EPW)], tidx_all, sem_g0).wait()
    pltpu.make_async_copy(et_hbm.at[pl.ds(base0, EPW)], et_all, sem_g0).wait()
    lane = lax.iota(jnp.int32, L)

    head_v = (head_v0, head_v1)
    tail_v = (tail_v0, tail_v1)
    sem_g = (sem_g0, sem_g1)

    def io(i, b):
        off = i * CHUNK
        pltpu.make_async_copy(
            zp_hbm.at[hidx_all.at[pl.ds(off, CHUNK)]], head_v[b], sem_g[b]).start()
        pltpu.make_async_copy(
            zp_hbm.at[tidx_all.at[pl.ds(off, CHUNK)]], tail_v[b], sem_g[b]).start()

    def compute(i, b):
        pltpu.make_async_copy(
            zp_hbm.at[hidx_all.at[pl.ds(0, CHUNK)]], head_v[b], sem_g[b]).wait()
        pltpu.make_async_copy(
            zp_hbm.at[hidx_all.at[pl.ds(0, CHUNK)]], tail_v[b], sem_g[b]).wait()

        def grp_body(g, _):
            et_grp = et_all[pl.ds(i * CHUNK + g * L, L)]
            accs = []
            for j in range(L):
                jj = _BREV[j]
                e = g * L + jj
                rb = et_grp[jj] * H
                acc = jnp.zeros((L,), jnp.float32)
                for m in range(2):
                    hr = head_v[b][e, pl.ds(m * 32, 32)]
                    hi = head_v[b][e, pl.ds(H + m * 32, 32)]
                    tr = tail_v[b][e, pl.ds(m * 32, 32)]
                    ti = tail_v[b][e, pl.ds(H + m * 32, 32)]
                    rr = rel_v[pl.ds(rb + m * 32, 32)]
                    ri = reli_v[pl.ds(rb + m * 32, 32)]
                    av = hr * rr - hi * ri
                    bv = hr * ri + hi * rr
                    ae, ao = plsc.unpack(av, format=plsc.PackFormat.INTERLEAVED)
                    be, bo = plsc.unpack(bv, format=plsc.PackFormat.INTERLEAVED)
                    te, to = plsc.unpack(tr, format=plsc.PackFormat.INTERLEAVED)
                    ue, uo = plsc.unpack(ti, format=plsc.PackFormat.INTERLEAVED)
                    acc = acc + ae * te + ao * to + be * ue + bo * uo
                accs.append(acc)
            vecs = accs
            for d in (8, 4, 2, 1):
                idx = lane ^ d
                msk = (lane & d) != 0
                folded = [v + jnp.take(v, idx)
                          for v in vecs]
                vecs = [jnp.where(msk, folded[2 * i + 1], folded[2 * i])
                        for i in range(len(folded) // 2)]
            scores_v[pl.ds(i * CHUNK + g * L, L)] = vecs[0]
            return 0

        lax.fori_loop(0, GROUPS, grp_body, 0)

    io(0, 0)

    def pair_body(p, _):
        i = 1 + 2 * p
        io(i, 1)
        compute(i - 1, 0)
        io(i + 1, 0)
        compute(i, 1)
        return 0

    lax.fori_loop(0, (NCHUNK - 1) // 2, pair_body, 0)
    compute(NCHUNK - 1, 0)
    pltpu.sync_copy(scores_v, out_hbm.at[pl.ds(base0, EPW)])


_BREV = [0, 8, 4, 12, 2, 10, 6, 14, 1, 9, 5, 13, 3, 11, 7, 15]


def kernel(z, edge_index, edge_type, rel_emb, rel_emb_imag):
    hidx = edge_index[0].astype(jnp.int32)
    tidx = edge_index[1].astype(jnp.int32)
    et = edge_type.astype(jnp.int32)
    zp = z.astype(jnp.bfloat16)
    mesh = plsc.VectorSubcoreMesh(
        core_axis_name="c", subcore_axis_name="s", num_cores=NC, num_subcores=NS
    )
    run = pl.kernel(
        _score_body,
        out_type=jax.ShapeDtypeStruct((NUM_EDGES,), jnp.float32),
        mesh=mesh,
        compiler_params=pltpu.CompilerParams(needs_layout_passes=False, use_tc_tiling_on_sc=False),
        scratch_types=[
            pltpu.VMEM((NUM_REL * H,), jnp.bfloat16),
            pltpu.VMEM((NUM_REL * H,), jnp.bfloat16),
            pltpu.VMEM((EPW,), jnp.int32),
            pltpu.VMEM((EPW,), jnp.int32),
            pltpu.VMEM((CHUNK, ZD), jnp.bfloat16),
            pltpu.VMEM((CHUNK, ZD), jnp.bfloat16),
            pltpu.VMEM((CHUNK, ZD), jnp.bfloat16),
            pltpu.VMEM((CHUNK, ZD), jnp.bfloat16),
            pltpu.VMEM((EPW,), jnp.int32),
            pltpu.VMEM((EPW,), jnp.float32),
            pltpu.SemaphoreType.DMA,
            pltpu.SemaphoreType.DMA,
        ],
    )
    relp = rel_emb.astype(jnp.bfloat16).reshape(-1)
    relip = rel_emb_imag.astype(jnp.bfloat16).reshape(-1)
    return run(zp, hidx, tidx, et, relp, relip)
